# baseline (speedup 1.0000x reference)
import functools

import jax
import jax.numpy as jnp
from jax import lax
from jax.experimental import pallas as pl
from jax.experimental.pallas import tpu as pltpu
from jax.experimental.pallas import tpu_sc as plsc

EMB = 64
NC, NS, L = 2, 16, 16
NW = NC * NS


def _sc_gather_pairs(user_ids, artist_ids, pg_u, pg_a, pm_u, pm_a):
    B = user_ids.shape[0]
    b_per_w = B // NW
    CH = 128
    n_ch = b_per_w // CH
    mesh = plsc.VectorSubcoreMesh(core_axis_name="c", subcore_axis_name="s")

    @functools.partial(
        pl.kernel,
        out_type=[jax.ShapeDtypeStruct((B, 2 * EMB), jnp.float32)] * 4,
        mesh=mesh,
        scratch_types=[
            pltpu.VMEM((b_per_w,), jnp.int32),
            pltpu.VMEM((b_per_w,), jnp.int32),
            pltpu.VMEM((b_per_w,), jnp.int32),
            pltpu.VMEM((b_per_w,), jnp.int32),
            pltpu.VMEM((CH, 2 * EMB), jnp.float32),
            pltpu.VMEM((CH, 2 * EMB), jnp.float32),
            pltpu.SemaphoreType.DMA,
            pltpu.SemaphoreType.DMA,
        ],
    )
    def gather_kernel(uid, aid, gu, ga, mu, ma, o_gu, o_ga, o_mu, o_ma,
                      idx_u, idx_a, pidx_u, pidx_a, buf0, buf1, sem0, sem1):
        wid = lax.axis_index("s") * NC + lax.axis_index("c")
        base = wid * b_per_w

        pltpu.sync_copy(uid.at[pl.ds(base, b_per_w)], idx_u)
        pltpu.sync_copy(aid.at[pl.ds(base, b_per_w)], idx_a)
        for g in range(b_per_w // L):
            sl = pl.ds(g * L, L)
            pidx_u[sl] = lax.shift_right_logical(idx_u[sl], 1)
            pidx_a[sl] = lax.shift_right_logical(idx_a[sl], 1)

        def fire(task, buf):
            table, pidx, _, c = task
            rows, sem = buf
            pltpu.async_copy(table.at[pidx.at[pl.ds(c * CH, CH)]], rows, sem)

        def finish(task, buf):
            table, _, out, c = task
            rows, sem = buf
            pltpu.make_async_copy(table.at[pl.ds(0, CH)], rows, sem).wait()
            pltpu.sync_copy(rows, out.at[pl.ds(base + c * CH, CH)])

        tasks = [(t, piv, o, c)
                 for (t, piv, o) in ((gu, pidx_u, o_gu), (ga, pidx_a, o_ga),
                                     (mu, pidx_u, o_mu), (ma, pidx_a, o_ma))
                 for c in range(n_ch)]
        bufs = [(buf0, sem0), (buf1, sem1)]
        for k, task in enumerate(tasks):
            if k >= 2:
                finish(tasks[k - 2], bufs[k % 2])
            fire(task, bufs[k % 2])
        finish(tasks[-2], bufs[len(tasks) % 2])
        finish(tasks[-1], bufs[(len(tasks) + 1) % 2])

    return gather_kernel(user_ids, artist_ids, pg_u, pg_a, pm_u, pm_a)


def _tc_mlp(pr_gu, pr_ga, pr_mu, pr_ma, par_u, par_a,
            W1, b1, W2, b2, W3, b3, Wf, bf):
    B = pr_gu.shape[0]
    BB = 2048
    w1u = W1[:, :EMB].T
    w1a = W1[:, EMB:].T
    w2t = W2.T
    w3t = W3.T
    wfg = Wf[:, :EMB]
    wfh = Wf[:, EMB:]
    b1r = b1.reshape(1, -1)
    b2r = b2.reshape(1, -1)
    b3r = b3.reshape(1, -1)
    bfr = bf.reshape(1, 1)

    def body(pgu, pga, pmu, pma, pu, pa, w1u_r, w1a_r, w2_r, w3_r,
             wfg_r, wfh_r, b1_r, b2_r, b3_r, bf_r, out_r):
        su = pu[...] == 1
        sa = pa[...] == 1
        gu = jnp.where(su, pgu[:, EMB:], pgu[:, :EMB])
        ga = jnp.where(sa, pga[:, EMB:], pga[:, :EMB])
        mu = jnp.where(su, pmu[:, EMB:], pmu[:, :EMB])
        ma = jnp.where(sa, pma[:, EMB:], pma[:, :EMB])
        dot = functools.partial(jnp.dot, preferred_element_type=jnp.float32)
        h = jnp.maximum(dot(mu, w1u_r[...]) + dot(ma, w1a_r[...])
                        + b1_r[...], 0.0)
        h = jnp.maximum(dot(h, w2_r[...]) + b2_r[...], 0.0)
        h = jnp.maximum(dot(h, w3_r[...]) + b3_r[...], 0.0)
        g = jnp.sum(gu * ga * wfg_r[...], axis=1, keepdims=True)
        m = jnp.sum(h * wfh_r[...], axis=1, keepdims=True)
        out_r[...] = jax.nn.sigmoid(g + m + bf_r[...])

    full = lambda a: pl.BlockSpec(a.shape, lambda i: (0, 0))
    pblk = pl.BlockSpec((BB, 2 * EMB), lambda i: (i, 0))
    iblk = pl.BlockSpec((BB, 1), lambda i: (i, 0))
    out = pl.pallas_call(
        body,
        grid=(B // BB,),
        in_specs=[pblk, pblk, pblk, pblk, iblk, iblk,
                  full(w1u), full(w1a), full(w2t), full(w3t),
                  full(wfg), full(wfh), full(b1r), full(b2r), full(b3r),
                  full(bfr)],
        out_specs=pl.BlockSpec((BB, 1), lambda i: (i, 0)),
        out_shape=jax.ShapeDtypeStruct((B, 1), jnp.float32),
    )(pr_gu, pr_ga, pr_mu, pr_ma, par_u, par_a, w1u, w1a, w2t, w3t,
      wfg, wfh, b1r, b2r, b3r, bfr)
    return out[:, 0]


def kernel(user_ids, artist_ids, gmf_user, gmf_artist, mlp_user, mlp_artist,
           W1, b1, W2, b2, W3, b3, Wf, bf):
    pg_u = gmf_user.reshape(-1, 2 * EMB)
    pg_a = gmf_artist.reshape(-1, 2 * EMB)
    pm_u = mlp_user.reshape(-1, 2 * EMB)
    pm_a = mlp_artist.reshape(-1, 2 * EMB)
    pr_gu, pr_ga, pr_mu, pr_ma = _sc_gather_pairs(
        user_ids, artist_ids, pg_u, pg_a, pm_u, pm_a)
    par_u = jnp.bitwise_and(user_ids, 1).reshape(-1, 1)
    par_a = jnp.bitwise_and(artist_ids, 1).reshape(-1, 1)
    return _tc_mlp(pr_gu, pr_ga, pr_mu, pr_ma, par_u, par_a,
                   W1, b1, W2, b2, W3, b3, Wf, bf)
